# baseline (speedup 1.0000x reference)
import jax
import jax.numpy as jnp
from jax.experimental import pallas as pl
from jax.experimental.pallas import tpu as pltpu

_BT = 512


def _body(adj_ref, fea_ref, W1_ref, b1_ref, W23_ref, b23_ref, eps_ref,
          Wd1T_ref, bd1_ref, Wd2T_ref, bd2_ref, gw_ref,
          acc_ref, s1_ref, x_ref, s23_ref, z_ref, accm_ref):
    t = pl.program_id(0)
    u = t % 64
    i = u // 8
    k = u % 8
    E = eps_ref.shape[1]

    @pl.when(t == 0)
    def _init():
        s1_ref[...] = jnp.dot(fea_ref[...], W1_ref[...],
                              preferred_element_type=jnp.float32)
        acc_ref[...] = jnp.zeros_like(acc_ref)

    @pl.when(k == 0)
    def _reset():
        accm_ref[...] = jnp.zeros_like(accm_ref)

    adj_tile = adj_ref[...]

    @pl.when(t < 64)
    def _phase1():
        accm_ref[...] += jnp.dot(adj_tile, s1_ref[pl.ds(k * _BT, _BT), :],
                                 preferred_element_type=jnp.float32)

        @pl.when(k == 7)
        def _():
            xblk = jax.nn.sigmoid(accm_ref[...] + b1_ref[...])
            x_ref[pl.ds(i * _BT, _BT), :] = xblk
            s23_ref[pl.ds(i * _BT, _BT), :] = jnp.dot(
                xblk, W23_ref[...], preferred_element_type=jnp.float32)

    @pl.when((t >= 64) & (t < 128))
    def _phase2():
        accm_ref[...] += jnp.dot(adj_tile, s23_ref[pl.ds(k * _BT, _BT), :],
                                 preferred_element_type=jnp.float32)

        @pl.when(k == 7)
        def _():
            ml = accm_ref[...] + b23_ref[...]
            mu = ml[:, :E]
            logvar = ml[:, E:]
            std = jnp.exp(0.5 * logvar)
            ev = std * std
            zblk = eps_ref[pl.ds(i * _BT, _BT), :] * std + mu
            z_ref[pl.ds(i * _BT, _BT), :] = zblk
            kld = -0.5 * jnp.sum(1.0 + logvar - mu * mu - ev)
            h = jax.nn.sigmoid(
                jnp.dot(zblk, Wd1T_ref[...],
                        preferred_element_type=jnp.float32) + bd1_ref[...])
            recon = jnp.dot(h, Wd2T_ref[...],
                            preferred_element_type=jnp.float32) + bd2_ref[...]
            fea_blk = fea_ref[pl.ds(i * _BT, _BT), :]
            fb = jnp.sum(jnp.maximum(recon, 0.0) - recon * fea_blk
                         + jnp.log1p(jnp.exp(-jnp.abs(recon))))
            acc_ref[...] += (kld + fb).reshape(1, 1)

    @pl.when(t >= 128)
    def _phase3():
        zi = z_ref[pl.ds(i * _BT, _BT), :]
        zj = z_ref[pl.ds(k * _BT, _BT), :]
        r = jax.lax.dot_general(zi, zj, (((1,), (1,)), ((), ())),
                                preferred_element_type=jnp.float32)
        sp = jnp.maximum(r, 0.0) + jnp.log(1.0 + jnp.exp(-jnp.abs(r)))
        term1 = jnp.sum(adj_tile * sp)
        a2 = adj_tile * adj_tile
        term2 = jnp.sum(zi * jnp.dot(a2, zj,
                                     preferred_element_type=jnp.float32))
        acc_ref[...] += gw_ref[...] * (term1 - term2)


def kernel(fea, fea_adj, adj, global_weight, W1, b1, W2, b2, W3, b3,
           Wd1, bd1, Wd2, bd2):
    del fea_adj
    N, F = fea.shape
    R = W1.shape[1]
    E = W2.shape[1]

    b1r = b1.reshape(1, R)
    W23 = jnp.concatenate([W2, W3], axis=1)
    b23 = jnp.concatenate([b2, b3]).reshape(1, 2 * E)
    Wd1T = Wd1.T
    bd1r = bd1.reshape(1, R)
    Wd2T = Wd2.T
    bd2r = bd2.reshape(1, F)
    eps = jax.random.normal(jax.random.key(42), (N, E), dtype=jnp.float32)
    gw = global_weight.reshape(1, 1)

    acc = pl.pallas_call(
        _body,
        grid=(192,),
        in_specs=[
            pl.BlockSpec((_BT, _BT),
                         lambda t: ((t % 64) // 8, (t % 64) % 8)),
            pl.BlockSpec((N, F), lambda t: (0, 0)),
            pl.BlockSpec((F, R), lambda t: (0, 0)),
            pl.BlockSpec((1, R), lambda t: (0, 0)),
            pl.BlockSpec((R, 2 * E), lambda t: (0, 0)),
            pl.BlockSpec((1, 2 * E), lambda t: (0, 0)),
            pl.BlockSpec((N, E), lambda t: (0, 0)),
            pl.BlockSpec((E, R), lambda t: (0, 0)),
            pl.BlockSpec((1, R), lambda t: (0, 0)),
            pl.BlockSpec((R, F), lambda t: (0, 0)),
            pl.BlockSpec((1, F), lambda t: (0, 0)),
            pl.BlockSpec((1, 1), lambda t: (0, 0)),
        ],
        out_specs=pl.BlockSpec((1, 1), lambda t: (0, 0)),
        out_shape=jax.ShapeDtypeStruct((1, 1), jnp.float32),
        scratch_shapes=[
            pltpu.VMEM((N, R), jnp.float32),
            pltpu.VMEM((N, R), jnp.float32),
            pltpu.VMEM((N, 2 * E), jnp.float32),
            pltpu.VMEM((N, E), jnp.float32),
            pltpu.VMEM((_BT, R), jnp.float32),
        ],
        compiler_params=pltpu.CompilerParams(
            dimension_semantics=("arbitrary",)),
    )(adj, fea, W1, b1r, W23, b23, eps, Wd1T, bd1r, Wd2T, bd2r, gw)

    return acc[0, 0]


# speedup vs baseline: 1.6419x; 1.6419x over previous
import jax
import jax.numpy as jnp
from jax.experimental import pallas as pl
from jax.experimental.pallas import tpu as pltpu

_BM = 512


def _p1_body(adj_ref, fea_ref, W1_ref, b1_ref, W23_ref, s23_ref, s1_ref):
    @pl.when(pl.program_id(0) == 0)
    def _():
        s1_ref[...] = jnp.dot(fea_ref[...], W1_ref[...],
                              preferred_element_type=jnp.float32)

    x = jax.nn.sigmoid(
        jnp.dot(adj_ref[...], s1_ref[...],
                preferred_element_type=jnp.float32) + b1_ref[...])
    s23_ref[...] = jnp.dot(x, W23_ref[...],
                           preferred_element_type=jnp.float32)


def _p2_body(adj_ref, s23_ref, b23_ref, eps_ref, fea_ref,
             Wd1T_ref, bd1_ref, Wd2T_ref, bd2_ref,
             z_ref, acc_ref):
    E = eps_ref.shape[1]

    @pl.when(pl.program_id(0) == 0)
    def _():
        acc_ref[...] = jnp.zeros_like(acc_ref)

    ml = jnp.dot(adj_ref[...], s23_ref[...],
                 preferred_element_type=jnp.float32) + b23_ref[...]
    mu = ml[:, :E]
    logvar = ml[:, E:]
    std = jnp.exp(0.5 * logvar)
    zblk = eps_ref[...] * std + mu
    z_ref[...] = zblk
    kld = -0.5 * jnp.sum(1.0 + logvar - mu * mu - jnp.exp(logvar))
    h = jax.nn.sigmoid(
        jnp.dot(zblk, Wd1T_ref[...],
                preferred_element_type=jnp.float32) + bd1_ref[...])
    recon = jnp.dot(h, Wd2T_ref[...],
                    preferred_element_type=jnp.float32) + bd2_ref[...]
    fea_blk = fea_ref[...]
    fb = jnp.sum(jnp.maximum(recon, 0.0) - recon * fea_blk
                 + jnp.log1p(jnp.exp(-jnp.abs(recon))))
    acc_ref[...] += (kld + fb).reshape(1, 1)


def _p3_body(adj_ref, z_ref, gw_ref, pacc_ref, out_ref):
    i = pl.program_id(0)

    @pl.when(i == 0)
    def _():
        out_ref[...] = pacc_ref[...]

    a = adj_ref[...]
    zi = z_ref[pl.ds(i * _BM, _BM), :]
    r = jax.lax.dot_general(zi, z_ref[...], (((1,), (1,)), ((), ())),
                            preferred_element_type=jnp.float32)
    sp = jnp.maximum(r, 0.0) + jnp.log1p(jnp.exp(-jnp.abs(r)))
    term1 = jnp.sum(a * sp)
    a2 = a * a
    term2 = jnp.sum(zi * jnp.dot(a2, z_ref[...],
                                 preferred_element_type=jnp.float32))
    out_ref[...] += gw_ref[...] * (term1 - term2)


def kernel(fea, fea_adj, adj, global_weight, W1, b1, W2, b2, W3, b3,
           Wd1, bd1, Wd2, bd2):
    del fea_adj
    N, F = fea.shape
    R = W1.shape[1]
    E = W2.shape[1]
    nblk = N // _BM

    b1r = b1.reshape(1, R)
    W23 = jnp.concatenate([W2, W3], axis=1)
    b23 = jnp.concatenate([b2, b3]).reshape(1, 2 * E)
    Wd1T = Wd1.T
    bd1r = bd1.reshape(1, R)
    Wd2T = Wd2.T
    bd2r = bd2.reshape(1, F)
    eps = jax.random.normal(jax.random.key(42), (N, E), dtype=jnp.float32)
    gw = global_weight.reshape(1, 1)

    s23 = pl.pallas_call(
        _p1_body,
        grid=(nblk,),
        in_specs=[
            pl.BlockSpec((_BM, N), lambda i: (i, 0)),
            pl.BlockSpec((N, F), lambda i: (0, 0)),
            pl.BlockSpec((F, R), lambda i: (0, 0)),
            pl.BlockSpec((1, R), lambda i: (0, 0)),
            pl.BlockSpec((R, 2 * E), lambda i: (0, 0)),
        ],
        out_specs=pl.BlockSpec((_BM, 2 * E), lambda i: (i, 0)),
        out_shape=jax.ShapeDtypeStruct((N, 2 * E), jnp.float32),
        scratch_shapes=[pltpu.VMEM((N, R), jnp.float32)],
        compiler_params=pltpu.CompilerParams(
            dimension_semantics=("arbitrary",)),
    )(adj, fea, W1, b1r, W23)

    z, acc2 = pl.pallas_call(
        _p2_body,
        grid=(nblk,),
        in_specs=[
            pl.BlockSpec((_BM, N), lambda i: (i, 0)),
            pl.BlockSpec((N, 2 * E), lambda i: (0, 0)),
            pl.BlockSpec((1, 2 * E), lambda i: (0, 0)),
            pl.BlockSpec((_BM, E), lambda i: (i, 0)),
            pl.BlockSpec((_BM, F), lambda i: (i, 0)),
            pl.BlockSpec((E, R), lambda i: (0, 0)),
            pl.BlockSpec((1, R), lambda i: (0, 0)),
            pl.BlockSpec((R, F), lambda i: (0, 0)),
            pl.BlockSpec((1, F), lambda i: (0, 0)),
        ],
        out_specs=[
            pl.BlockSpec((_BM, E), lambda i: (i, 0)),
            pl.BlockSpec((1, 1), lambda i: (0, 0)),
        ],
        out_shape=[
            jax.ShapeDtypeStruct((N, E), jnp.float32),
            jax.ShapeDtypeStruct((1, 1), jnp.float32),
        ],
        compiler_params=pltpu.CompilerParams(
            dimension_semantics=("arbitrary",)),
    )(adj, s23, b23, eps, fea, Wd1T, bd1r, Wd2T, bd2r)

    acc = pl.pallas_call(
        _p3_body,
        grid=(nblk,),
        in_specs=[
            pl.BlockSpec((_BM, N), lambda i: (i, 0)),
            pl.BlockSpec((N, E), lambda i: (0, 0)),
            pl.BlockSpec((1, 1), lambda i: (0, 0)),
            pl.BlockSpec((1, 1), lambda i: (0, 0)),
        ],
        out_specs=pl.BlockSpec((1, 1), lambda i: (0, 0)),
        out_shape=jax.ShapeDtypeStruct((1, 1), jnp.float32),
        compiler_params=pltpu.CompilerParams(
            dimension_semantics=("arbitrary",)),
    )(adj, z, gw, acc2)

    return acc[0, 0]


# speedup vs baseline: 1.6574x; 1.0094x over previous
import jax
import jax.numpy as jnp
from jax.experimental import pallas as pl
from jax.experimental.pallas import tpu as pltpu

_BM = 512
_NB = 8


def _body(adj_ref, fea_ref, W1_ref, b1_ref, W23_ref, b23_ref, eps_ref,
          Wd1T_ref, bd1_ref, Wd2T_ref, bd2_ref, gw_ref,
          out_ref, s1_ref, s23_ref, z_ref):
    t = pl.program_id(0)
    i = jax.lax.rem(t, _NB)
    E = z_ref.shape[1]
    a = adj_ref[...]

    @pl.when(t == 0)
    def _init():
        s1_ref[...] = jnp.dot(fea_ref[...], W1_ref[...],
                              preferred_element_type=jnp.float32)
        out_ref[...] = jnp.zeros_like(out_ref)

    @pl.when(t < _NB)
    def _phase1():
        x = jax.nn.sigmoid(
            jnp.dot(a, s1_ref[...],
                    preferred_element_type=jnp.float32) + b1_ref[...])
        s23_ref[pl.ds(i * _BM, _BM), :] = jnp.dot(
            x, W23_ref[...], preferred_element_type=jnp.float32)

    @pl.when((t >= _NB) & (t < 2 * _NB))
    def _phase2():
        ml = jnp.dot(a, s23_ref[...],
                     preferred_element_type=jnp.float32) + b23_ref[...]
        mu = ml[:, :E]
        logvar = ml[:, E:]
        std = jnp.exp(0.5 * logvar)
        zblk = eps_ref[pl.ds(i * _BM, _BM), :] * std + mu
        z_ref[pl.ds(i * _BM, _BM), :] = zblk
        kld = -0.5 * jnp.sum(1.0 + logvar - mu * mu - jnp.exp(logvar))
        h = jax.nn.sigmoid(
            jnp.dot(zblk, Wd1T_ref[...],
                    preferred_element_type=jnp.float32) + bd1_ref[...])
        recon = jnp.dot(h, Wd2T_ref[...],
                        preferred_element_type=jnp.float32) + bd2_ref[...]
        fea_blk = fea_ref[pl.ds(i * _BM, _BM), :]
        fb = jnp.sum(jnp.maximum(recon, 0.0) - recon * fea_blk
                     + jnp.log1p(jnp.exp(-jnp.abs(recon))))
        out_ref[...] += (kld + fb).reshape(1, 1)

    @pl.when(t >= 2 * _NB)
    def _phase3():
        zi = z_ref[pl.ds(i * _BM, _BM), :]
        r = jax.lax.dot_general(zi, z_ref[...], (((1,), (1,)), ((), ())),
                                preferred_element_type=jnp.float32)
        sp = jnp.maximum(r, 0.0) + jnp.log1p(jnp.exp(-jnp.abs(r)))
        term1 = jnp.sum(a * sp)
        a2 = a * a
        term2 = jnp.sum(zi * jnp.dot(a2, z_ref[...],
                                     preferred_element_type=jnp.float32))
        out_ref[...] += gw_ref[...] * (term1 - term2)


def kernel(fea, fea_adj, adj, global_weight, W1, b1, W2, b2, W3, b3,
           Wd1, bd1, Wd2, bd2):
    del fea_adj
    N, F = fea.shape
    R = W1.shape[1]
    E = W2.shape[1]

    b1r = b1.reshape(1, R)
    W23 = jnp.concatenate([W2, W3], axis=1)
    b23 = jnp.concatenate([b2, b3]).reshape(1, 2 * E)
    Wd1T = Wd1.T
    bd1r = bd1.reshape(1, R)
    Wd2T = Wd2.T
    bd2r = bd2.reshape(1, F)
    eps = jax.random.normal(jax.random.key(42), (N, E), dtype=jnp.float32)
    gw = global_weight.reshape(1, 1)

    acc = pl.pallas_call(
        _body,
        grid=(3 * _NB,),
        in_specs=[
            pl.BlockSpec((_BM, N), lambda t: (t % _NB, 0)),
            pl.BlockSpec((N, F), lambda t: (0, 0)),
            pl.BlockSpec((F, R), lambda t: (0, 0)),
            pl.BlockSpec((1, R), lambda t: (0, 0)),
            pl.BlockSpec((R, 2 * E), lambda t: (0, 0)),
            pl.BlockSpec((1, 2 * E), lambda t: (0, 0)),
            pl.BlockSpec((N, E), lambda t: (0, 0)),
            pl.BlockSpec((E, R), lambda t: (0, 0)),
            pl.BlockSpec((1, R), lambda t: (0, 0)),
            pl.BlockSpec((R, F), lambda t: (0, 0)),
            pl.BlockSpec((1, F), lambda t: (0, 0)),
            pl.BlockSpec((1, 1), lambda t: (0, 0)),
        ],
        out_specs=pl.BlockSpec((1, 1), lambda t: (0, 0)),
        out_shape=jax.ShapeDtypeStruct((1, 1), jnp.float32),
        scratch_shapes=[
            pltpu.VMEM((N, R), jnp.float32),
            pltpu.VMEM((N, 2 * E), jnp.float32),
            pltpu.VMEM((N, E), jnp.float32),
        ],
        compiler_params=pltpu.CompilerParams(
            dimension_semantics=("arbitrary",)),
    )(adj, fea, W1, b1r, W23, b23, eps, Wd1T, bd1r, Wd2T, bd2r, gw)

    return acc[0, 0]


# speedup vs baseline: 1.8104x; 1.0923x over previous
import jax
import jax.numpy as jnp
from jax.experimental import pallas as pl
from jax.experimental.pallas import tpu as pltpu

_BM = 512
_NB = 8


def _body(adj_ref, fea_ref, W1_ref, b1_ref, W23_ref, b23_ref, eps_ref,
          Wd1T_ref, bd1_ref, Wd2T_ref, bd2_ref, gw_ref,
          out_ref, s1_ref, s23_ref, z_ref):
    t = pl.program_id(0)
    i = jax.lax.rem(t, _NB)
    E = z_ref.shape[1]
    a = adj_ref[...]

    @pl.when(t == 0)
    def _init():
        s1_ref[...] = jnp.dot(fea_ref[...], W1_ref[...],
                              preferred_element_type=jnp.float32)
        out_ref[...] = jnp.zeros_like(out_ref)

    @pl.when(t < _NB)
    def _phase1():
        x = jax.nn.sigmoid(
            jnp.dot(a, s1_ref[...],
                    preferred_element_type=jnp.float32) + b1_ref[...])
        s23_ref[pl.ds(i * _BM, _BM), :] = jnp.dot(
            x, W23_ref[...], preferred_element_type=jnp.float32)

    @pl.when((t >= _NB) & (t < 2 * _NB))
    def _phase2():
        ml = jnp.dot(a, s23_ref[...],
                     preferred_element_type=jnp.float32) + b23_ref[...]
        mu = ml[:, :E]
        logvar = ml[:, E:]
        std = jnp.exp(0.5 * logvar)
        zblk = eps_ref[pl.ds(i * _BM, _BM), :] * std + mu
        z_ref[pl.ds(i * _BM, _BM), :] = zblk
        kld = -0.5 * jnp.sum(1.0 + logvar - mu * mu - jnp.exp(logvar))
        h = jax.nn.sigmoid(
            jnp.dot(zblk, Wd1T_ref[...],
                    preferred_element_type=jnp.float32) + bd1_ref[...])
        recon = jnp.dot(h, Wd2T_ref[...],
                        preferred_element_type=jnp.float32) + bd2_ref[...]
        fea_blk = fea_ref[pl.ds(i * _BM, _BM), :]
        fb = jnp.sum(jnp.maximum(recon, 0.0) - recon * fea_blk
                     + jnp.log1p(jnp.exp(-jnp.abs(recon))))
        out_ref[...] += (kld + fb).reshape(1, 1)

    @pl.when(t >= 2 * _NB)
    def _phase3():
        zi = z_ref[pl.ds(i * _BM, _BM), :]
        r = jax.lax.dot_general(zi, z_ref[...], (((1,), (1,)), ((), ())),
                                preferred_element_type=jnp.float32)
        u = r * jnp.float32(1.4426950408889634)
        nu = jax.lax.bitcast_convert_type(
            jax.lax.bitcast_convert_type(u, jnp.uint32)
            | jnp.uint32(0x80000000), jnp.float32)
        g = jnp.maximum(u, 0.0) + jnp.log2(1.0 + jnp.exp2(nu))
        term = jnp.sum(a * (jnp.float32(0.6931471805599453) * g - a * r))
        out_ref[...] += gw_ref[...] * term


def kernel(fea, fea_adj, adj, global_weight, W1, b1, W2, b2, W3, b3,
           Wd1, bd1, Wd2, bd2):
    del fea_adj
    N, F = fea.shape
    R = W1.shape[1]
    E = W2.shape[1]

    b1r = b1.reshape(1, R)
    W23 = jnp.concatenate([W2, W3], axis=1)
    b23 = jnp.concatenate([b2, b3]).reshape(1, 2 * E)
    Wd1T = Wd1.T
    bd1r = bd1.reshape(1, R)
    Wd2T = Wd2.T
    bd2r = bd2.reshape(1, F)
    eps = jax.random.normal(jax.random.key(42), (N, E), dtype=jnp.float32)
    gw = global_weight.reshape(1, 1)

    acc = pl.pallas_call(
        _body,
        grid=(3 * _NB,),
        in_specs=[
            pl.BlockSpec((_BM, N), lambda t: (t % _NB, 0)),
            pl.BlockSpec((N, F), lambda t: (0, 0)),
            pl.BlockSpec((F, R), lambda t: (0, 0)),
            pl.BlockSpec((1, R), lambda t: (0, 0)),
            pl.BlockSpec((R, 2 * E), lambda t: (0, 0)),
            pl.BlockSpec((1, 2 * E), lambda t: (0, 0)),
            pl.BlockSpec((N, E), lambda t: (0, 0)),
            pl.BlockSpec((E, R), lambda t: (0, 0)),
            pl.BlockSpec((1, R), lambda t: (0, 0)),
            pl.BlockSpec((R, F), lambda t: (0, 0)),
            pl.BlockSpec((1, F), lambda t: (0, 0)),
            pl.BlockSpec((1, 1), lambda t: (0, 0)),
        ],
        out_specs=pl.BlockSpec((1, 1), lambda t: (0, 0)),
        out_shape=jax.ShapeDtypeStruct((1, 1), jnp.float32),
        scratch_shapes=[
            pltpu.VMEM((N, R), jnp.float32),
            pltpu.VMEM((N, 2 * E), jnp.float32),
            pltpu.VMEM((N, E), jnp.float32),
        ],
        compiler_params=pltpu.CompilerParams(
            dimension_semantics=("arbitrary",)),
    )(adj, fea, W1, b1r, W23, b23, eps, Wd1T, bd1r, Wd2T, bd2r, gw)

    return acc[0, 0]
